# baseline (speedup 1.0000x reference)
import functools

import jax
import jax.numpy as jnp
import numpy as np
from jax.experimental import pallas as pl
from jax.experimental.pallas import tpu as pltpu

FEAT_STRIDE = 16
POST_NMS_TOP_N = 300
_H = 64
_W = 64
_ROWS = 288
_LANES = 128


def _mk(ws, hs, x_ctr, y_ctr):
    ws = np.asarray(ws, dtype=np.float64).reshape(-1, 1)
    hs = np.asarray(hs, dtype=np.float64).reshape(-1, 1)
    return np.hstack((x_ctr - 0.5 * (ws - 1), y_ctr - 0.5 * (hs - 1),
                      x_ctr + 0.5 * (ws - 1), y_ctr + 0.5 * (hs - 1)))


def _gen_anchors(base_size=16, ratios=(0.5, 1.0, 2.0), scales=(8, 16, 32),
                 time_dim=(16,)):
    ratios = np.array(ratios)
    scales = np.array(scales)
    x_ctr = 0.5 * (base_size - 1)
    y_ctr = 0.5 * (base_size - 1)
    size = float(base_size * base_size)
    size_ratios = size / ratios
    ws = np.round(np.sqrt(size_ratios))
    hs = np.round(ws * ratios)
    ratio_anchors = _mk(ws, hs, x_ctr, y_ctr)
    all_a = []
    for a in ratio_anchors:
        w = a[2] - a[0] + 1.0
        h = a[3] - a[1] + 1.0
        xc = a[0] + 0.5 * (w - 1)
        yc = a[1] + 0.5 * (h - 1)
        all_a.append(_mk(w * scales, h * scales, xc, yc))
    a2d = np.vstack(all_a)
    out = []
    for t in time_dim:
        n = a2d.shape[0]
        out.append(np.hstack((a2d[:, 0:2], np.zeros((n, 1)), a2d[:, 2:4],
                              np.full((n, 1), float(t) - 1.0))))
    return np.vstack(out).astype(np.float32)


_ANC = _gen_anchors()
_A = _ANC.shape[0]
_Z1 = float(_ANC[0, 2])
_Z2 = float(_ANC[0, 5])


def _anchor_planes():
    w9 = _ANC[:, 3] - _ANC[:, 0] + 1.0
    h9 = _ANC[:, 4] - _ANC[:, 1] + 1.0
    cx9 = _ANC[:, 0] + 0.5 * w9
    cy9 = _ANC[:, 1] + 0.5 * h9
    k = np.arange(_H * _W)
    sx = (k % _W) * float(FEAT_STRIDE)
    sy = (k // _W) * float(FEAT_STRIDE)
    W = np.broadcast_to(w9[None, :], (_H * _W, _A)).reshape(-1)
    H = np.broadcast_to(h9[None, :], (_H * _W, _A)).reshape(-1)
    CX = (cx9[None, :] + sx[:, None]).reshape(-1)
    CY = (cy9[None, :] + sy[:, None]).reshape(-1)
    shape = (_ROWS, _LANES)
    return (W.reshape(shape).astype(np.float32),
            H.reshape(shape).astype(np.float32),
            CX.reshape(shape).astype(np.float32),
            CY.reshape(shape).astype(np.float32))


_AW, _AH, _ACX, _ACY = _anchor_planes()


def _proposal_kernel(im_ref, sc_ref, dx_ref, dy_ref, dw_ref, dh_ref,
                     aw_ref, ah_ref, acx_ref, acy_ref, out_ref, buf):
    b = pl.program_id(0)
    xmax = im_ref[b, 1] - 1.0
    ymax = im_ref[b, 0] - 1.0
    bcol = jnp.float32(b)

    buf[...] = sc_ref[0]

    idx2d = jax.lax.broadcasted_iota(jnp.int32, (_ROWS, _LANES), 0) * _LANES \
        + jax.lax.broadcasted_iota(jnp.int32, (_ROWS, _LANES), 1)
    lane = jax.lax.broadcasted_iota(jnp.int32, (1, _LANES), 1)

    def body(j, _):
        data = buf[...]
        m = jnp.max(data)
        cand = jnp.min(jnp.where(data == m, idx2d, jnp.int32(2**30)))
        r = cand // _LANES
        c = cand % _LANES
        sel = lane == c

        def gat(ref):
            return jnp.sum(jnp.where(sel, ref[0, pl.ds(r, 1), :], 0.0))

        dx, dy, dw, dh = gat(dx_ref), gat(dy_ref), gat(dw_ref), gat(dh_ref)
        aw, ah, acx, acy = gat(aw_ref), gat(ah_ref), gat(acx_ref), gat(acy_ref)

        pcx = dx * aw + acx
        pcy = dy * ah + acy
        pw = jnp.exp(dw) * aw
        ph = jnp.exp(dh) * ah
        x1 = jnp.clip(pcx - 0.5 * pw, 0.0, xmax)
        y1 = jnp.clip(pcy - 0.5 * ph, 0.0, ymax)
        x2 = jnp.clip(pcx + 0.5 * pw, 0.0, xmax)
        y2 = jnp.clip(pcy + 0.5 * ph, 0.0, ymax)

        row = jnp.full((1, _LANES), 0.0, dtype=jnp.float32)
        for li, v in enumerate((bcol, x1, y1, jnp.float32(_Z1),
                                x2, y2, jnp.float32(_Z2), m)):
            row = jnp.where(lane == li, v, row)
        out_ref[0, pl.ds(j, 1), :] = row

        buf[pl.ds(r, 1), :] = jnp.where(sel, -jnp.inf,
                                        buf[pl.ds(r, 1), :])
        return 0

    jax.lax.fori_loop(0, POST_NMS_TOP_N, body, 0)


@jax.jit
def _run(scores, bbox_frame, im_info):
    B = scores.shape[0]
    sc = jnp.transpose(scores[:, _A:, :, :], (0, 2, 3, 1)).reshape(
        B, _ROWS, _LANES)
    bb = jnp.transpose(bbox_frame, (0, 2, 3, 1)).reshape(B, _ROWS * _LANES, 4)
    planes = [bb[..., d].reshape(B, _ROWS, _LANES) for d in range(4)]
    consts = [jnp.asarray(p) for p in (_AW, _AH, _ACX, _ACY)]

    data_spec = pl.BlockSpec((1, _ROWS, _LANES), lambda b: (b, 0, 0))
    const_spec = pl.BlockSpec((1, _ROWS, _LANES), lambda b: (0, 0, 0))
    out = pl.pallas_call(
        _proposal_kernel,
        grid=(B,),
        in_specs=[pl.BlockSpec(memory_space=pltpu.SMEM)]
        + [data_spec] * 5
        + [const_spec] * 4,
        out_specs=pl.BlockSpec((1, 304, _LANES), lambda b: (b, 0, 0)),
        out_shape=jax.ShapeDtypeStruct((B, 304, _LANES), jnp.float32),
        scratch_shapes=[pltpu.VMEM((_ROWS, _LANES), jnp.float32)],
    )(im_info, sc, *planes, *[c.reshape(1, _ROWS, _LANES) for c in consts])
    return out[:, :POST_NMS_TOP_N, :8]


def kernel(scores, bbox_frame, im_info, time_dim):
    return _run(scores, bbox_frame, im_info)
